# baseline (speedup 1.0000x reference)
import dataclasses
import functools

import jax
import jax.numpy as jnp
from jax import lax
from jax.experimental import pallas as pl
from jax.experimental.pallas import tpu as pltpu
from jax.experimental.pallas import tpu_sc as plsc

LANES = 16
E_CHUNK = 64
N_WORKERS = 32
NEG_SLOPE = 0.2
ROW_BLK = 1024
TAB_W = 128


def _pre_body(node_ref, wl_ref, wr_ref, wlin_ref, att_ref, blin_ref,
              tab_ref, lin_ref, snum_ref, sden_ref):
    n = node_ref[...]
    xl = jnp.dot(n, wl_ref[...], preferred_element_type=jnp.float32)
    xr = jnp.dot(n, wr_ref[...], preferred_element_type=jnp.float32)
    lin = jnp.dot(n, wlin_ref[...], preferred_element_type=jnp.float32)
    tab_ref[...] = jnp.concatenate([xl, xr], axis=1)
    lin_ref[...] = lin + blin_ref[...]
    t = xl + xr
    leaky = jnp.maximum(t, NEG_SLOPE * t)
    w = jnp.exp(jnp.sum(leaky * att_ref[...], axis=1, keepdims=True))
    snum_ref[...] = xl * w
    sden_ref[...] = w


def _post_body(acc0_ref, acc1_ref, snum_ref, sden_ref, lin_ref, bias_ref,
               out_ref):
    out_ch = out_ref.shape[1]
    a0 = acc0_ref[...]
    a1 = acc1_ref[...]
    num = a0[:, :out_ch] + a1[:, :out_ch] + snum_ref[...]
    den = (a0[:, out_ch:out_ch + 1] + a1[:, out_ch:out_ch + 1]
           + sden_ref[...] + 1e-16)
    out_ref[...] = jnp.maximum(num / den + bias_ref[...] + lin_ref[...], 0.0)


def _make_sc_kernel(n_pad, n_acc, out_ch, chunks_per_worker):
    rows_per_sub = n_acc // 16
    cregs = out_ch // LANES

    def body(tab_hbm, src_hbm, dst_hbm, att_hbm, acc_out,
             sidx0, didx0, sidx1, didx1, sd0, sd1,
             a0, b0, a1, b1, att_v, lg_v, w_v, acc_sh,
             isem0, isem1, ssem0, ssem1):
        cid = lax.axis_index("c")
        sid = lax.axis_index("s")
        zero16 = jnp.zeros((LANES,), jnp.float32)
        lane0 = jnp.arange(LANES) == 0
        n_edges = chunks_per_worker * N_WORKERS * E_CHUNK
        last_base = n_edges - E_CHUNK

        @pl.loop(0, E_CHUNK)
        def _(i):
            for c in range(TAB_W // LANES):
                a0[i, pl.ds(c * LANES, LANES)] = zero16

        row0 = sid * rows_per_sub
        full_chunks = (rows_per_sub // E_CHUNK) * E_CHUNK
        tail = rows_per_sub - full_chunks

        @pl.loop(0, full_chunks, step=E_CHUNK)
        def _(r):
            pltpu.sync_copy(a0, acc_sh.at[pl.ds(row0 + r, E_CHUNK)])
        if tail:
            pltpu.sync_copy(a0.at[pl.ds(0, tail)],
                            acc_sh.at[pl.ds(row0 + full_chunks, tail)])

        pltpu.sync_copy(att_hbm, att_v)
        plsc.subcore_barrier()

        att_regs = [att_v[pl.ds(c * LANES, LANES)] for c in range(cregs)]
        wid = cid * 16 + sid
        base0 = wid * chunks_per_worker * E_CHUNK

        def chunk_base(t):
            return jnp.minimum(base0 + t * E_CHUNK, last_base)

        def issue_idx(t, sidx, didx, isem):
            base = chunk_base(t)
            pltpu.async_copy(src_hbm.at[pl.ds(base, E_CHUNK)], sidx, isem)
            pltpu.async_copy(dst_hbm.at[pl.ds(base, E_CHUNK)], didx, isem)

        def wait_idx(sidx, didx, isem):
            pltpu.make_async_copy(src_hbm.at[pl.ds(0, E_CHUNK)], sidx,
                                  isem).wait()
            pltpu.make_async_copy(dst_hbm.at[pl.ds(0, E_CHUNK)], didx,
                                  isem).wait()

        def issue_gathers(sidx, didx, a, b, isem):
            pltpu.async_copy(tab_hbm.at[sidx], a, isem)
            pltpu.async_copy(tab_hbm.at[didx], b, isem)

        def wait_gathers(sidx, didx, a, b, isem):
            pltpu.make_async_copy(tab_hbm.at[sidx], a, isem).wait()
            pltpu.make_async_copy(tab_hbm.at[didx], b, isem).wait()

        def wait_scatter(sd, msg, ssem):
            pltpu.make_async_copy(msg, acc_sh.at[sd], ssem).wait()

        def compute_chunk(a_v, b_v):
            @pl.loop(0, E_CHUNK, unroll=4)
            def _(e):
                s = None
                for c in range(cregs):
                    sl = pl.ds(c * LANES, LANES)
                    slr = pl.ds(out_ch + c * LANES, LANES)
                    t_ = a_v[e, sl] + b_v[e, slr]
                    l_ = jnp.maximum(t_, NEG_SLOPE * t_) * att_regs[c]
                    s = l_ if s is None else s + l_
                logit = jnp.sum(s)
                eidx = jnp.full((LANES,), e, jnp.int32)
                plsc.store_scatter(lg_v, [eidx], zero16 + logit, mask=lane0)

            @pl.loop(0, E_CHUNK, step=LANES)
            def _(j):
                w_v[pl.ds(j, LANES)] = jnp.exp(lg_v[pl.ds(j, LANES)])

            @pl.loop(0, E_CHUNK, unroll=4)
            def _(e):
                eidx = jnp.full((LANES,), e, jnp.int32)
                wb = plsc.load_gather(w_v, [eidx])
                for c in range(cregs):
                    sl = pl.ds(c * LANES, LANES)
                    a_v[e, sl] = a_v[e, sl] * wb
                a_v[e, pl.ds(out_ch, LANES)] = wb

        def stash_didx(didx, sd):
            @pl.loop(0, E_CHUNK, step=LANES)
            def _(j):
                sd[pl.ds(j, LANES)] = didx[pl.ds(j, LANES)]

        issue_idx(0, sidx0, didx0, isem0)
        issue_idx(1, sidx1, didx1, isem1)
        wait_idx(sidx0, didx0, isem0)
        issue_gathers(sidx0, didx0, a0, b0, isem0)

        @pl.loop(0, chunks_per_worker, step=2)
        def _(t):
            wait_idx(sidx1, didx1, isem1)
            wait_gathers(sidx0, didx0, a0, b0, isem0)

            @pl.when(t > 0)
            def _():
                wait_scatter(sd1, a1, ssem1)
            issue_gathers(sidx1, didx1, a1, b1, isem1)

            stash_didx(didx0, sd0)
            issue_idx(t + 2, sidx0, didx0, isem0)
            compute_chunk(a0, b0)
            pltpu.async_copy(a0, acc_sh.at[sd0], ssem0, add=True)

            wait_gathers(sidx1, didx1, a1, b1, isem1)
            stash_didx(didx1, sd1)
            issue_idx(t + 3, sidx1, didx1, isem1)
            compute_chunk(a1, b1)
            pltpu.async_copy(a1, acc_sh.at[sd1], ssem1, add=True)

            wait_idx(sidx0, didx0, isem0)
            wait_scatter(sd0, a0, ssem0)
            issue_gathers(sidx0, didx0, a0, b0, isem0)

        wait_gathers(sidx0, didx0, a0, b0, isem0)
        wait_idx(sidx1, didx1, isem1)
        wait_scatter(sd1, a1, ssem1)

        plsc.subcore_barrier()
        pltpu.sync_copy(acc_sh.at[pl.ds(row0, rows_per_sub)],
                        acc_out.at[cid].at[pl.ds(row0, rows_per_sub)])

    mesh = plsc.VectorSubcoreMesh(core_axis_name="c", subcore_axis_name="s")
    cp = pltpu.CompilerParams()
    if "needs_layout_passes" in pltpu.CompilerParams.__dataclass_fields__:
        cp = dataclasses.replace(cp, needs_layout_passes=False)
    return pl.kernel(
        body,
        compiler_params=cp,
        out_type=jax.ShapeDtypeStruct((2, n_pad, TAB_W), jnp.float32),
        mesh=mesh,
        scratch_types=[
            pltpu.VMEM((E_CHUNK,), jnp.int32),
            pltpu.VMEM((E_CHUNK,), jnp.int32),
            pltpu.VMEM((E_CHUNK,), jnp.int32),
            pltpu.VMEM((E_CHUNK,), jnp.int32),
            pltpu.VMEM((E_CHUNK,), jnp.int32),
            pltpu.VMEM((E_CHUNK,), jnp.int32),
            pltpu.VMEM((E_CHUNK, TAB_W), jnp.float32),
            pltpu.VMEM((E_CHUNK, TAB_W), jnp.float32),
            pltpu.VMEM((E_CHUNK, TAB_W), jnp.float32),
            pltpu.VMEM((E_CHUNK, TAB_W), jnp.float32),
            pltpu.VMEM((out_ch,), jnp.float32),
            pltpu.VMEM((E_CHUNK,), jnp.float32),
            pltpu.VMEM((E_CHUNK,), jnp.float32),
            pltpu.VMEM_SHARED((n_acc, TAB_W), jnp.float32),
            pltpu.SemaphoreType.DMA,
            pltpu.SemaphoreType.DMA,
            pltpu.SemaphoreType.DMA,
            pltpu.SemaphoreType.DMA,
        ],
    )


def kernel(node, edge_index, W_l, W_r, att, bias_gnn, W_lin, b_lin):
    n, in_ch = node.shape
    out_ch = W_l.shape[0]
    e = edge_index.shape[1]

    grain = 2 * N_WORKERS * E_CHUNK
    e_pad = ((e + grain - 1) // grain) * grain
    chunks_per_worker = e_pad // (N_WORKERS * E_CHUNK)
    n_pad = ((max(n + 1, ROW_BLK) + (16 * E_CHUNK) - 1)
             // (16 * E_CHUNK)) * (16 * E_CHUNK)
    n_acc = ((n + 1 + 16 * 8 - 1) // (16 * 8)) * (16 * 8)

    node_pad = jnp.concatenate(
        [node, jnp.zeros((n_pad - n, in_ch), jnp.float32)], axis=0)
    src = edge_index[0].astype(jnp.int32)
    dst = edge_index[1].astype(jnp.int32)
    pad = e_pad - e
    src_p = jnp.concatenate([src, jnp.zeros((pad,), jnp.int32)])
    dst_p = jnp.concatenate([dst, jnp.full((pad,), n, jnp.int32)])

    att2 = att.reshape(1, out_ch)
    blin2 = b_lin.reshape(1, out_ch)
    bias2 = bias_gnn.reshape(1, out_ch)

    grid = (n_pad // ROW_BLK,)
    row_spec = pl.BlockSpec((ROW_BLK, out_ch), lambda i: (i, 0))
    one_spec = pl.BlockSpec((ROW_BLK, 1), lambda i: (i, 0))
    wide_spec = pl.BlockSpec((ROW_BLK, TAB_W), lambda i: (i, 0))
    full_spec = lambda r, c: pl.BlockSpec((r, c), lambda i: (0, 0))

    tab, lin, snum, sden = pl.pallas_call(
        _pre_body,
        grid=grid,
        in_specs=[
            pl.BlockSpec((ROW_BLK, in_ch), lambda i: (i, 0)),
            full_spec(in_ch, out_ch),
            full_spec(in_ch, out_ch),
            full_spec(in_ch, out_ch),
            full_spec(1, out_ch),
            full_spec(1, out_ch),
        ],
        out_specs=[wide_spec, row_spec, row_spec, one_spec],
        out_shape=[
            jax.ShapeDtypeStruct((n_pad, TAB_W), jnp.float32),
            jax.ShapeDtypeStruct((n_pad, out_ch), jnp.float32),
            jax.ShapeDtypeStruct((n_pad, out_ch), jnp.float32),
            jax.ShapeDtypeStruct((n_pad, 1), jnp.float32),
        ],
    )(node_pad, W_l.T, W_r.T, W_lin.T, att2, blin2)

    edge_fn = _make_sc_kernel(n_pad, n_acc, out_ch, chunks_per_worker)
    acc_all = edge_fn(tab, src_p, dst_p, att)

    out = pl.pallas_call(
        _post_body,
        grid=grid,
        in_specs=[wide_spec, wide_spec, row_spec, one_spec, row_spec,
                  full_spec(1, out_ch)],
        out_specs=row_spec,
        out_shape=jax.ShapeDtypeStruct((n_pad, out_ch), jnp.float32),
    )(acc_all[0], acc_all[1], snum, sden, lin, bias2)

    return out[:n]


# speedup vs baseline: 1.2456x; 1.2456x over previous
import dataclasses
import functools

import jax
import jax.numpy as jnp
from jax import lax
from jax.experimental import pallas as pl
from jax.experimental.pallas import tpu as pltpu
from jax.experimental.pallas import tpu_sc as plsc

LANES = 16
E_CHUNK = 64
N_WORKERS = 32
NEG_SLOPE = 0.2
ROW_BLK = 1024
TAB_W = 128


def _pre_body(node_ref, wl_ref, wr_ref, wlin_ref, att_ref, blin_ref,
              tab_ref, lin_ref, snum_ref, sden_ref):
    n = node_ref[...]
    xl = jnp.dot(n, wl_ref[...], preferred_element_type=jnp.float32)
    xr = jnp.dot(n, wr_ref[...], preferred_element_type=jnp.float32)
    lin = jnp.dot(n, wlin_ref[...], preferred_element_type=jnp.float32)
    tab_ref[...] = jnp.concatenate([xl, xr], axis=1)
    lin_ref[...] = lin + blin_ref[...]
    t = xl + xr
    leaky = jnp.maximum(t, NEG_SLOPE * t)
    w = jnp.exp(jnp.sum(leaky * att_ref[...], axis=1, keepdims=True))
    snum_ref[...] = xl * w
    sden_ref[...] = w


def _post_body(acc0_ref, acc1_ref, snum_ref, sden_ref, lin_ref, bias_ref,
               out_ref):
    out_ch = out_ref.shape[1]
    a0 = acc0_ref[...]
    a1 = acc1_ref[...]
    num = a0[:, :out_ch] + a1[:, :out_ch] + snum_ref[...]
    den = (a0[:, out_ch:out_ch + 1] + a1[:, out_ch:out_ch + 1]
           + sden_ref[...] + 1e-16)
    out_ref[...] = jnp.maximum(num / den + bias_ref[...] + lin_ref[...], 0.0)


def _make_sc_kernel(n_pad, n_acc, out_ch, chunks_per_worker):
    rows_per_sub = n_acc // 16
    cregs = out_ch // LANES

    def body(tab_hbm, src_hbm, dst_hbm, att_hbm, acc_out,
             sidx0, didx0, sidx1, didx1, sd0, sd1,
             a0, b0, a1, b1, att_v, lg_v, w_v, acc_sh,
             isem0, isem1, ssem0, ssem1):
        cid = lax.axis_index("c")
        sid = lax.axis_index("s")
        zero16 = jnp.zeros((LANES,), jnp.float32)
        lane0 = jnp.arange(LANES) == 0
        n_edges = chunks_per_worker * N_WORKERS * E_CHUNK
        last_base = n_edges - E_CHUNK

        @pl.loop(0, E_CHUNK)
        def _(i):
            for c in range(TAB_W // LANES):
                a0[i, pl.ds(c * LANES, LANES)] = zero16

        row0 = sid * rows_per_sub
        full_chunks = (rows_per_sub // E_CHUNK) * E_CHUNK
        tail = rows_per_sub - full_chunks

        @pl.loop(0, full_chunks, step=E_CHUNK)
        def _(r):
            pltpu.sync_copy(a0, acc_sh.at[pl.ds(row0 + r, E_CHUNK)])
        if tail:
            pltpu.sync_copy(a0.at[pl.ds(0, tail)],
                            acc_sh.at[pl.ds(row0 + full_chunks, tail)])

        pltpu.sync_copy(att_hbm, att_v)
        plsc.subcore_barrier()

        att_regs = [att_v[pl.ds(c * LANES, LANES)] for c in range(cregs)]
        wid = cid * 16 + sid
        base0 = wid * chunks_per_worker * E_CHUNK

        def chunk_base(t):
            return jnp.minimum(base0 + t * E_CHUNK, last_base)

        def issue_idx(t, sidx, didx, isem):
            base = chunk_base(t)
            pltpu.async_copy(src_hbm.at[pl.ds(base, E_CHUNK)], sidx, isem)
            pltpu.async_copy(dst_hbm.at[pl.ds(base, E_CHUNK)], didx, isem)

        def wait_idx(sidx, didx, isem):
            pltpu.make_async_copy(src_hbm.at[pl.ds(0, E_CHUNK)], sidx,
                                  isem).wait()
            pltpu.make_async_copy(dst_hbm.at[pl.ds(0, E_CHUNK)], didx,
                                  isem).wait()

        def issue_gathers(sidx, didx, a, b, isem):
            pltpu.async_copy(tab_hbm.at[sidx], a, isem)
            pltpu.async_copy(tab_hbm.at[didx], b, isem)

        def wait_gathers(sidx, didx, a, b, isem):
            pltpu.make_async_copy(tab_hbm.at[sidx], a, isem).wait()
            pltpu.make_async_copy(tab_hbm.at[didx], b, isem).wait()

        def wait_scatter(sd, msg, ssem):
            pltpu.make_async_copy(msg, acc_sh.at[sd], ssem).wait()

        def compute_chunk(a_v, b_v):
            @plsc.parallel_loop(0, E_CHUNK, 1, unroll=4)
            def _(e):
                s = None
                for c in range(cregs):
                    sl = pl.ds(c * LANES, LANES)
                    slr = pl.ds(out_ch + c * LANES, LANES)
                    t_ = a_v[e, sl] + b_v[e, slr]
                    l_ = jnp.maximum(t_, NEG_SLOPE * t_) * att_regs[c]
                    s = l_ if s is None else s + l_
                logit = jnp.sum(s)
                eidx = jnp.full((LANES,), e, jnp.int32)
                plsc.store_scatter(lg_v, [eidx], zero16 + logit, mask=lane0)

            @pl.loop(0, E_CHUNK, step=LANES)
            def _(j):
                w_v[pl.ds(j, LANES)] = jnp.exp(lg_v[pl.ds(j, LANES)])

            @plsc.parallel_loop(0, E_CHUNK, 1, unroll=4)
            def _(e):
                eidx = jnp.full((LANES,), e, jnp.int32)
                wb = plsc.load_gather(w_v, [eidx])
                for c in range(cregs):
                    sl = pl.ds(c * LANES, LANES)
                    a_v[e, sl] = a_v[e, sl] * wb
                a_v[e, pl.ds(out_ch, LANES)] = wb

        def stash_didx(didx, sd):
            @pl.loop(0, E_CHUNK, step=LANES)
            def _(j):
                sd[pl.ds(j, LANES)] = didx[pl.ds(j, LANES)]

        issue_idx(0, sidx0, didx0, isem0)
        issue_idx(1, sidx1, didx1, isem1)
        wait_idx(sidx0, didx0, isem0)
        issue_gathers(sidx0, didx0, a0, b0, isem0)

        @pl.loop(0, chunks_per_worker, step=2)
        def _(t):
            wait_idx(sidx1, didx1, isem1)
            wait_gathers(sidx0, didx0, a0, b0, isem0)

            @pl.when(t > 0)
            def _():
                wait_scatter(sd1, a1, ssem1)
            issue_gathers(sidx1, didx1, a1, b1, isem1)

            stash_didx(didx0, sd0)
            issue_idx(t + 2, sidx0, didx0, isem0)
            compute_chunk(a0, b0)
            pltpu.async_copy(a0, acc_sh.at[sd0], ssem0, add=True)

            wait_gathers(sidx1, didx1, a1, b1, isem1)
            stash_didx(didx1, sd1)
            issue_idx(t + 3, sidx1, didx1, isem1)
            compute_chunk(a1, b1)
            pltpu.async_copy(a1, acc_sh.at[sd1], ssem1, add=True)

            wait_idx(sidx0, didx0, isem0)
            wait_scatter(sd0, a0, ssem0)
            issue_gathers(sidx0, didx0, a0, b0, isem0)

        wait_gathers(sidx0, didx0, a0, b0, isem0)
        wait_idx(sidx1, didx1, isem1)
        wait_scatter(sd1, a1, ssem1)

        plsc.subcore_barrier()
        pltpu.sync_copy(acc_sh.at[pl.ds(row0, rows_per_sub)],
                        acc_out.at[cid].at[pl.ds(row0, rows_per_sub)])

    mesh = plsc.VectorSubcoreMesh(core_axis_name="c", subcore_axis_name="s")
    cp = pltpu.CompilerParams()
    if "needs_layout_passes" in pltpu.CompilerParams.__dataclass_fields__:
        cp = dataclasses.replace(cp, needs_layout_passes=False)
    return pl.kernel(
        body,
        compiler_params=cp,
        out_type=jax.ShapeDtypeStruct((2, n_pad, TAB_W), jnp.float32),
        mesh=mesh,
        scratch_types=[
            pltpu.VMEM((E_CHUNK,), jnp.int32),
            pltpu.VMEM((E_CHUNK,), jnp.int32),
            pltpu.VMEM((E_CHUNK,), jnp.int32),
            pltpu.VMEM((E_CHUNK,), jnp.int32),
            pltpu.VMEM((E_CHUNK,), jnp.int32),
            pltpu.VMEM((E_CHUNK,), jnp.int32),
            pltpu.VMEM((E_CHUNK, TAB_W), jnp.float32),
            pltpu.VMEM((E_CHUNK, TAB_W), jnp.float32),
            pltpu.VMEM((E_CHUNK, TAB_W), jnp.float32),
            pltpu.VMEM((E_CHUNK, TAB_W), jnp.float32),
            pltpu.VMEM((out_ch,), jnp.float32),
            pltpu.VMEM((E_CHUNK,), jnp.float32),
            pltpu.VMEM((E_CHUNK,), jnp.float32),
            pltpu.VMEM_SHARED((n_acc, TAB_W), jnp.float32),
            pltpu.SemaphoreType.DMA,
            pltpu.SemaphoreType.DMA,
            pltpu.SemaphoreType.DMA,
            pltpu.SemaphoreType.DMA,
        ],
    )


def kernel(node, edge_index, W_l, W_r, att, bias_gnn, W_lin, b_lin):
    n, in_ch = node.shape
    out_ch = W_l.shape[0]
    e = edge_index.shape[1]

    grain = 2 * N_WORKERS * E_CHUNK
    e_pad = ((e + grain - 1) // grain) * grain
    chunks_per_worker = e_pad // (N_WORKERS * E_CHUNK)
    n_pad = ((max(n + 1, ROW_BLK) + (16 * E_CHUNK) - 1)
             // (16 * E_CHUNK)) * (16 * E_CHUNK)
    n_acc = ((n + 1 + 16 * 8 - 1) // (16 * 8)) * (16 * 8)

    node_pad = jnp.concatenate(
        [node, jnp.zeros((n_pad - n, in_ch), jnp.float32)], axis=0)
    src = edge_index[0].astype(jnp.int32)
    dst = edge_index[1].astype(jnp.int32)
    pad = e_pad - e
    src_p = jnp.concatenate([src, jnp.zeros((pad,), jnp.int32)])
    dst_p = jnp.concatenate([dst, jnp.full((pad,), n, jnp.int32)])

    att2 = att.reshape(1, out_ch)
    blin2 = b_lin.reshape(1, out_ch)
    bias2 = bias_gnn.reshape(1, out_ch)

    grid = (n_pad // ROW_BLK,)
    row_spec = pl.BlockSpec((ROW_BLK, out_ch), lambda i: (i, 0))
    one_spec = pl.BlockSpec((ROW_BLK, 1), lambda i: (i, 0))
    wide_spec = pl.BlockSpec((ROW_BLK, TAB_W), lambda i: (i, 0))
    full_spec = lambda r, c: pl.BlockSpec((r, c), lambda i: (0, 0))

    tab, lin, snum, sden = pl.pallas_call(
        _pre_body,
        grid=grid,
        in_specs=[
            pl.BlockSpec((ROW_BLK, in_ch), lambda i: (i, 0)),
            full_spec(in_ch, out_ch),
            full_spec(in_ch, out_ch),
            full_spec(in_ch, out_ch),
            full_spec(1, out_ch),
            full_spec(1, out_ch),
        ],
        out_specs=[wide_spec, row_spec, row_spec, one_spec],
        out_shape=[
            jax.ShapeDtypeStruct((n_pad, TAB_W), jnp.float32),
            jax.ShapeDtypeStruct((n_pad, out_ch), jnp.float32),
            jax.ShapeDtypeStruct((n_pad, out_ch), jnp.float32),
            jax.ShapeDtypeStruct((n_pad, 1), jnp.float32),
        ],
    )(node_pad, W_l.T, W_r.T, W_lin.T, att2, blin2)

    edge_fn = _make_sc_kernel(n_pad, n_acc, out_ch, chunks_per_worker)
    acc_all = edge_fn(tab, src_p, dst_p, att)

    out = pl.pallas_call(
        _post_body,
        grid=grid,
        in_specs=[wide_spec, wide_spec, row_spec, one_spec, row_spec,
                  full_spec(1, out_ch)],
        out_specs=row_spec,
        out_shape=jax.ShapeDtypeStruct((n_pad, out_ch), jnp.float32),
    )(acc_all[0], acc_all[1], snum, sden, lin, bias2)

    return out[:n]


# speedup vs baseline: 1.2478x; 1.0018x over previous
import dataclasses
import functools

import jax
import jax.numpy as jnp
from jax import lax
from jax.experimental import pallas as pl
from jax.experimental.pallas import tpu as pltpu
from jax.experimental.pallas import tpu_sc as plsc

LANES = 16
E_CHUNK = 64
N_WORKERS = 32
NEG_SLOPE = 0.2
ROW_BLK = 1024
TAB_W = 128


def _pre_body(node_ref, wl_ref, wr_ref, wlin_ref, att_ref, blin_ref,
              tab_ref, lin_ref, snum_ref, sden_ref):
    n = node_ref[...]
    xl = jnp.dot(n, wl_ref[...], preferred_element_type=jnp.float32)
    xr = jnp.dot(n, wr_ref[...], preferred_element_type=jnp.float32)
    lin = jnp.dot(n, wlin_ref[...], preferred_element_type=jnp.float32)
    tab_ref[...] = jnp.concatenate([xl, xr], axis=1)
    lin_ref[...] = lin + blin_ref[...]
    t = xl + xr
    leaky = jnp.maximum(t, NEG_SLOPE * t)
    w = jnp.exp(jnp.sum(leaky * att_ref[...], axis=1, keepdims=True))
    snum_ref[...] = xl * w
    sden_ref[...] = w


def _post_body(acc0_ref, acc1_ref, snum_ref, sden_ref, lin_ref, bias_ref,
               out_ref):
    out_ch = out_ref.shape[1]
    a0 = acc0_ref[...]
    a1 = acc1_ref[...]
    num = a0[:, :out_ch] + a1[:, :out_ch] + snum_ref[...]
    den = (a0[:, out_ch:out_ch + 1] + a1[:, out_ch:out_ch + 1]
           + sden_ref[...] + 1e-16)
    out_ref[...] = jnp.maximum(num / den + bias_ref[...] + lin_ref[...], 0.0)


def _make_sc_kernel(n_pad, n_acc, out_ch, chunks_per_worker):
    rows_per_sub = n_acc // 16
    cregs = out_ch // LANES

    def body(tab_hbm, src_hbm, dst_hbm, att_hbm, acc_out,
             sidx0, didx0, sidx1, didx1, sd0, sd1,
             a0, b0, a1, b1, att_v, lg_v, w_v, acc_sh,
             isem0, isem1, ssem0, ssem1):
        cid = lax.axis_index("c")
        sid = lax.axis_index("s")
        zero16 = jnp.zeros((LANES,), jnp.float32)
        lane0 = jnp.arange(LANES) == 0
        n_edges = chunks_per_worker * N_WORKERS * E_CHUNK
        last_base = n_edges - E_CHUNK

        @pl.loop(0, E_CHUNK)
        def _(i):
            for c in range(TAB_W // LANES):
                a0[i, pl.ds(c * LANES, LANES)] = zero16

        row0 = sid * rows_per_sub
        full_chunks = (rows_per_sub // E_CHUNK) * E_CHUNK
        tail = rows_per_sub - full_chunks

        @pl.loop(0, full_chunks, step=E_CHUNK)
        def _(r):
            pltpu.sync_copy(a0, acc_sh.at[pl.ds(row0 + r, E_CHUNK)])
        if tail:
            pltpu.sync_copy(a0.at[pl.ds(0, tail)],
                            acc_sh.at[pl.ds(row0 + full_chunks, tail)])

        pltpu.sync_copy(att_hbm, att_v)
        plsc.subcore_barrier()

        att_regs = [att_v[pl.ds(c * LANES, LANES)] for c in range(cregs)]
        wid = cid * 16 + sid
        base0 = wid * chunks_per_worker * E_CHUNK

        def chunk_base(t):
            return jnp.minimum(base0 + t * E_CHUNK, last_base)

        def issue_idx(t, sidx, didx, isem):
            base = chunk_base(t)
            pltpu.async_copy(src_hbm.at[pl.ds(base, E_CHUNK)], sidx, isem)
            pltpu.async_copy(dst_hbm.at[pl.ds(base, E_CHUNK)], didx, isem)

        def wait_idx(sidx, didx, isem):
            pltpu.make_async_copy(src_hbm.at[pl.ds(0, E_CHUNK)], sidx,
                                  isem).wait()
            pltpu.make_async_copy(dst_hbm.at[pl.ds(0, E_CHUNK)], didx,
                                  isem).wait()

        def issue_gathers(sidx, didx, a, b, isem):
            pltpu.async_copy(tab_hbm.at[sidx], a, isem)
            pltpu.async_copy(tab_hbm.at[didx], b, isem)

        def wait_gathers(sidx, didx, a, b, isem):
            pltpu.make_async_copy(tab_hbm.at[sidx], a, isem).wait()
            pltpu.make_async_copy(tab_hbm.at[didx], b, isem).wait()

        def wait_scatter(sd, msg, ssem):
            pltpu.make_async_copy(msg, acc_sh.at[sd], ssem).wait()

        def compute_chunk(a_v, b_v):
            @plsc.parallel_loop(0, E_CHUNK, 1, unroll=8)
            def _(e):
                s = None
                for c in range(cregs):
                    sl = pl.ds(c * LANES, LANES)
                    slr = pl.ds(out_ch + c * LANES, LANES)
                    t_ = a_v[e, sl] + b_v[e, slr]
                    l_ = jnp.maximum(t_, NEG_SLOPE * t_) * att_regs[c]
                    s = l_ if s is None else s + l_
                logit = jnp.sum(s)
                eidx = jnp.full((LANES,), e, jnp.int32)
                plsc.store_scatter(lg_v, [eidx], zero16 + logit, mask=lane0)

            @pl.loop(0, E_CHUNK, step=LANES)
            def _(j):
                w_v[pl.ds(j, LANES)] = jnp.exp(lg_v[pl.ds(j, LANES)])

            @plsc.parallel_loop(0, E_CHUNK, 1, unroll=8)
            def _(e):
                eidx = jnp.full((LANES,), e, jnp.int32)
                wb = plsc.load_gather(w_v, [eidx])
                for c in range(cregs):
                    sl = pl.ds(c * LANES, LANES)
                    a_v[e, sl] = a_v[e, sl] * wb
                a_v[e, pl.ds(out_ch, LANES)] = wb

        def stash_didx(didx, sd):
            @pl.loop(0, E_CHUNK, step=LANES)
            def _(j):
                sd[pl.ds(j, LANES)] = didx[pl.ds(j, LANES)]

        issue_idx(0, sidx0, didx0, isem0)
        issue_idx(1, sidx1, didx1, isem1)
        wait_idx(sidx0, didx0, isem0)
        issue_gathers(sidx0, didx0, a0, b0, isem0)

        @pl.loop(0, chunks_per_worker, step=2)
        def _(t):
            wait_idx(sidx1, didx1, isem1)
            wait_gathers(sidx0, didx0, a0, b0, isem0)

            @pl.when(t > 0)
            def _():
                wait_scatter(sd1, a1, ssem1)
            issue_gathers(sidx1, didx1, a1, b1, isem1)

            stash_didx(didx0, sd0)
            issue_idx(t + 2, sidx0, didx0, isem0)
            compute_chunk(a0, b0)
            pltpu.async_copy(a0, acc_sh.at[sd0], ssem0, add=True)

            wait_gathers(sidx1, didx1, a1, b1, isem1)
            stash_didx(didx1, sd1)
            issue_idx(t + 3, sidx1, didx1, isem1)
            compute_chunk(a1, b1)
            pltpu.async_copy(a1, acc_sh.at[sd1], ssem1, add=True)

            wait_idx(sidx0, didx0, isem0)
            wait_scatter(sd0, a0, ssem0)
            issue_gathers(sidx0, didx0, a0, b0, isem0)

        wait_gathers(sidx0, didx0, a0, b0, isem0)
        wait_idx(sidx1, didx1, isem1)
        wait_scatter(sd1, a1, ssem1)

        plsc.subcore_barrier()
        pltpu.sync_copy(acc_sh.at[pl.ds(row0, rows_per_sub)],
                        acc_out.at[cid].at[pl.ds(row0, rows_per_sub)])

    mesh = plsc.VectorSubcoreMesh(core_axis_name="c", subcore_axis_name="s")
    cp = pltpu.CompilerParams()
    if "needs_layout_passes" in pltpu.CompilerParams.__dataclass_fields__:
        cp = dataclasses.replace(cp, needs_layout_passes=False)
    return pl.kernel(
        body,
        compiler_params=cp,
        out_type=jax.ShapeDtypeStruct((2, n_pad, TAB_W), jnp.float32),
        mesh=mesh,
        scratch_types=[
            pltpu.VMEM((E_CHUNK,), jnp.int32),
            pltpu.VMEM((E_CHUNK,), jnp.int32),
            pltpu.VMEM((E_CHUNK,), jnp.int32),
            pltpu.VMEM((E_CHUNK,), jnp.int32),
            pltpu.VMEM((E_CHUNK,), jnp.int32),
            pltpu.VMEM((E_CHUNK,), jnp.int32),
            pltpu.VMEM((E_CHUNK, TAB_W), jnp.float32),
            pltpu.VMEM((E_CHUNK, TAB_W), jnp.float32),
            pltpu.VMEM((E_CHUNK, TAB_W), jnp.float32),
            pltpu.VMEM((E_CHUNK, TAB_W), jnp.float32),
            pltpu.VMEM((out_ch,), jnp.float32),
            pltpu.VMEM((E_CHUNK,), jnp.float32),
            pltpu.VMEM((E_CHUNK,), jnp.float32),
            pltpu.VMEM_SHARED((n_acc, TAB_W), jnp.float32),
            pltpu.SemaphoreType.DMA,
            pltpu.SemaphoreType.DMA,
            pltpu.SemaphoreType.DMA,
            pltpu.SemaphoreType.DMA,
        ],
    )


def kernel(node, edge_index, W_l, W_r, att, bias_gnn, W_lin, b_lin):
    n, in_ch = node.shape
    out_ch = W_l.shape[0]
    e = edge_index.shape[1]

    grain = 2 * N_WORKERS * E_CHUNK
    e_pad = ((e + grain - 1) // grain) * grain
    chunks_per_worker = e_pad // (N_WORKERS * E_CHUNK)
    n_pad = ((max(n + 1, ROW_BLK) + (16 * E_CHUNK) - 1)
             // (16 * E_CHUNK)) * (16 * E_CHUNK)
    n_acc = ((n + 1 + 16 * 8 - 1) // (16 * 8)) * (16 * 8)

    node_pad = jnp.concatenate(
        [node, jnp.zeros((n_pad - n, in_ch), jnp.float32)], axis=0)
    src = edge_index[0].astype(jnp.int32)
    dst = edge_index[1].astype(jnp.int32)
    pad = e_pad - e
    src_p = jnp.concatenate([src, jnp.zeros((pad,), jnp.int32)])
    dst_p = jnp.concatenate([dst, jnp.full((pad,), n, jnp.int32)])

    att2 = att.reshape(1, out_ch)
    blin2 = b_lin.reshape(1, out_ch)
    bias2 = bias_gnn.reshape(1, out_ch)

    grid = (n_pad // ROW_BLK,)
    row_spec = pl.BlockSpec((ROW_BLK, out_ch), lambda i: (i, 0))
    one_spec = pl.BlockSpec((ROW_BLK, 1), lambda i: (i, 0))
    wide_spec = pl.BlockSpec((ROW_BLK, TAB_W), lambda i: (i, 0))
    full_spec = lambda r, c: pl.BlockSpec((r, c), lambda i: (0, 0))

    tab, lin, snum, sden = pl.pallas_call(
        _pre_body,
        grid=grid,
        in_specs=[
            pl.BlockSpec((ROW_BLK, in_ch), lambda i: (i, 0)),
            full_spec(in_ch, out_ch),
            full_spec(in_ch, out_ch),
            full_spec(in_ch, out_ch),
            full_spec(1, out_ch),
            full_spec(1, out_ch),
        ],
        out_specs=[wide_spec, row_spec, row_spec, one_spec],
        out_shape=[
            jax.ShapeDtypeStruct((n_pad, TAB_W), jnp.float32),
            jax.ShapeDtypeStruct((n_pad, out_ch), jnp.float32),
            jax.ShapeDtypeStruct((n_pad, out_ch), jnp.float32),
            jax.ShapeDtypeStruct((n_pad, 1), jnp.float32),
        ],
    )(node_pad, W_l.T, W_r.T, W_lin.T, att2, blin2)

    edge_fn = _make_sc_kernel(n_pad, n_acc, out_ch, chunks_per_worker)
    acc_all = edge_fn(tab, src_p, dst_p, att)

    out = pl.pallas_call(
        _post_body,
        grid=grid,
        in_specs=[wide_spec, wide_spec, row_spec, one_spec, row_spec,
                  full_spec(1, out_ch)],
        out_specs=row_spec,
        out_shape=jax.ShapeDtypeStruct((n_pad, out_ch), jnp.float32),
    )(acc_all[0], acc_all[1], snum, sden, lin, bias2)

    return out[:n]


# speedup vs baseline: 1.3230x; 1.0603x over previous
import dataclasses
import functools

import jax
import jax.numpy as jnp
from jax import lax
from jax.experimental import pallas as pl
from jax.experimental.pallas import tpu as pltpu
from jax.experimental.pallas import tpu_sc as plsc

LANES = 16
E_CHUNK = 80
N_WORKERS = 32
NEG_SLOPE = 0.2
ROW_BLK = 1024
TAB_W = 128


def _pre_body(node_ref, wl_ref, wr_ref, wlin_ref, att_ref, blin_ref,
              tab_ref, lin_ref, snum_ref, sden_ref):
    n = node_ref[...]
    xl = jnp.dot(n, wl_ref[...], preferred_element_type=jnp.float32)
    xr = jnp.dot(n, wr_ref[...], preferred_element_type=jnp.float32)
    lin = jnp.dot(n, wlin_ref[...], preferred_element_type=jnp.float32)
    tab_ref[...] = jnp.concatenate([xl, xr], axis=1)
    lin_ref[...] = lin + blin_ref[...]
    t = xl + xr
    leaky = jnp.maximum(t, NEG_SLOPE * t)
    w = jnp.exp(jnp.sum(leaky * att_ref[...], axis=1, keepdims=True))
    snum_ref[...] = xl * w
    sden_ref[...] = w


def _post_body(acc0_ref, acc1_ref, snum_ref, sden_ref, lin_ref, bias_ref,
               out_ref):
    out_ch = out_ref.shape[1]
    a0 = acc0_ref[...]
    a1 = acc1_ref[...]
    num = a0[:, :out_ch] + a1[:, :out_ch] + snum_ref[...]
    den = (a0[:, out_ch:out_ch + 1] + a1[:, out_ch:out_ch + 1]
           + sden_ref[...] + 1e-16)
    out_ref[...] = jnp.maximum(num / den + bias_ref[...] + lin_ref[...], 0.0)


def _make_sc_kernel(n_pad, n_acc, out_ch, chunks_per_worker):
    rows_per_sub = n_acc // 16
    cregs = out_ch // LANES

    def body(tab_hbm, ei_hbm, att_hbm, acc_out,
             i0, i1, sd0, sd1,
             a0, b0, a1, b1, att_v, lg_v, w_v, acc_sh,
             isem0, isem1, ssem0, ssem1):
        cid = lax.axis_index("c")
        sid = lax.axis_index("s")
        zero16 = jnp.zeros((LANES,), jnp.float32)
        lane0 = jnp.arange(LANES) == 0
        n_chunks = chunks_per_worker * N_WORKERS

        @pl.loop(0, E_CHUNK)
        def _(i):
            for c in range(TAB_W // LANES):
                a0[i, pl.ds(c * LANES, LANES)] = zero16

        row0 = sid * rows_per_sub
        full_chunks = (rows_per_sub // E_CHUNK) * E_CHUNK
        tail = rows_per_sub - full_chunks

        @pl.loop(0, full_chunks, step=E_CHUNK)
        def _(r):
            pltpu.sync_copy(a0, acc_sh.at[pl.ds(row0 + r, E_CHUNK)])
        if tail:
            pltpu.sync_copy(a0.at[pl.ds(0, tail)],
                            acc_sh.at[pl.ds(row0 + full_chunks, tail)])

        pltpu.sync_copy(att_hbm, att_v)
        plsc.subcore_barrier()

        att_regs = [att_v[pl.ds(c * LANES, LANES)] for c in range(cregs)]
        wid = cid * 16 + sid
        cbase = wid * chunks_per_worker

        def issue_idx(t, ibuf, isem):
            ci = jnp.minimum(cbase + t, n_chunks - 1)
            pltpu.async_copy(ei_hbm.at[ci], ibuf, isem)

        def wait_idx(ibuf, isem):
            pltpu.make_async_copy(ei_hbm.at[0], ibuf, isem).wait()

        def issue_gathers(ibuf, a, b, isem):
            pltpu.async_copy(tab_hbm.at[ibuf.at[0]], a, isem)
            pltpu.async_copy(tab_hbm.at[ibuf.at[1]], b, isem)

        def wait_gathers(ibuf, a, b, isem):
            pltpu.make_async_copy(tab_hbm.at[ibuf.at[0]], a, isem).wait()
            pltpu.make_async_copy(tab_hbm.at[ibuf.at[1]], b, isem).wait()

        def wait_scatter(sd, msg, ssem):
            pltpu.make_async_copy(msg, acc_sh.at[sd], ssem).wait()

        def compute_chunk(a_v, b_v):
            @plsc.parallel_loop(0, E_CHUNK, 1, unroll=8)
            def _(e):
                s = None
                for c in range(cregs):
                    sl = pl.ds(c * LANES, LANES)
                    slr = pl.ds(out_ch + c * LANES, LANES)
                    t_ = a_v[e, sl] + b_v[e, slr]
                    l_ = jnp.maximum(t_, NEG_SLOPE * t_) * att_regs[c]
                    s = l_ if s is None else s + l_
                logit = jnp.sum(s)
                eidx = jnp.full((LANES,), e, jnp.int32)
                plsc.store_scatter(lg_v, [eidx], zero16 + logit, mask=lane0)

            @pl.loop(0, E_CHUNK, step=LANES)
            def _(j):
                w_v[pl.ds(j, LANES)] = jnp.exp(lg_v[pl.ds(j, LANES)])

            @plsc.parallel_loop(0, E_CHUNK, 1, unroll=8)
            def _(e):
                eidx = jnp.full((LANES,), e, jnp.int32)
                wb = plsc.load_gather(w_v, [eidx])
                for c in range(cregs):
                    sl = pl.ds(c * LANES, LANES)
                    a_v[e, sl] = a_v[e, sl] * wb
                a_v[e, pl.ds(out_ch, LANES)] = wb

        def stash_didx(ibuf, sd):
            @pl.loop(0, E_CHUNK, step=LANES)
            def _(j):
                sd[pl.ds(j, LANES)] = ibuf[1, pl.ds(j, LANES)]

        issue_idx(0, i0, isem0)
        issue_idx(1, i1, isem1)
        wait_idx(i0, isem0)
        issue_gathers(i0, a0, b0, isem0)

        @pl.loop(0, chunks_per_worker, step=2)
        def _(t):
            wait_idx(i1, isem1)
            wait_gathers(i0, a0, b0, isem0)

            @pl.when(t > 0)
            def _():
                wait_scatter(sd1, a1, ssem1)
            issue_gathers(i1, a1, b1, isem1)

            stash_didx(i0, sd0)
            issue_idx(t + 2, i0, isem0)
            compute_chunk(a0, b0)
            pltpu.async_copy(a0, acc_sh.at[sd0], ssem0, add=True)

            wait_gathers(i1, a1, b1, isem1)
            stash_didx(i1, sd1)
            issue_idx(t + 3, i1, isem1)
            compute_chunk(a1, b1)
            pltpu.async_copy(a1, acc_sh.at[sd1], ssem1, add=True)

            wait_idx(i0, isem0)
            wait_scatter(sd0, a0, ssem0)
            issue_gathers(i0, a0, b0, isem0)

        wait_gathers(i0, a0, b0, isem0)
        wait_idx(i1, isem1)
        wait_scatter(sd1, a1, ssem1)

        plsc.subcore_barrier()
        pltpu.sync_copy(acc_sh.at[pl.ds(row0, rows_per_sub)],
                        acc_out.at[cid].at[pl.ds(row0, rows_per_sub)])

    mesh = plsc.VectorSubcoreMesh(core_axis_name="c", subcore_axis_name="s")
    cp = pltpu.CompilerParams()
    if "needs_layout_passes" in pltpu.CompilerParams.__dataclass_fields__:
        cp = dataclasses.replace(cp, needs_layout_passes=False)
    return pl.kernel(
        body,
        compiler_params=cp,
        out_type=jax.ShapeDtypeStruct((2, n_pad, TAB_W), jnp.float32),
        mesh=mesh,
        scratch_types=[
            pltpu.VMEM((2, E_CHUNK), jnp.int32),
            pltpu.VMEM((2, E_CHUNK), jnp.int32),
            pltpu.VMEM((E_CHUNK,), jnp.int32),
            pltpu.VMEM((E_CHUNK,), jnp.int32),
            pltpu.VMEM((E_CHUNK, TAB_W), jnp.float32),
            pltpu.VMEM((E_CHUNK, TAB_W), jnp.float32),
            pltpu.VMEM((E_CHUNK, TAB_W), jnp.float32),
            pltpu.VMEM((E_CHUNK, TAB_W), jnp.float32),
            pltpu.VMEM((out_ch,), jnp.float32),
            pltpu.VMEM((E_CHUNK,), jnp.float32),
            pltpu.VMEM((E_CHUNK,), jnp.float32),
            pltpu.VMEM_SHARED((n_acc, TAB_W), jnp.float32),
            pltpu.SemaphoreType.DMA,
            pltpu.SemaphoreType.DMA,
            pltpu.SemaphoreType.DMA,
            pltpu.SemaphoreType.DMA,
        ],
    )


def kernel(node, edge_index, W_l, W_r, att, bias_gnn, W_lin, b_lin):
    n, in_ch = node.shape
    out_ch = W_l.shape[0]
    e = edge_index.shape[1]

    grain = 2 * N_WORKERS * E_CHUNK
    e_pad = ((e + grain - 1) // grain) * grain
    chunks_per_worker = e_pad // (N_WORKERS * E_CHUNK)
    n_pad = ((max(n + 1, ROW_BLK) + (16 * E_CHUNK) - 1)
             // (16 * E_CHUNK)) * (16 * E_CHUNK)
    n_acc = ((n + 1 + 16 * 8 - 1) // (16 * 8)) * (16 * 8)

    node_pad = jnp.concatenate(
        [node, jnp.zeros((n_pad - n, in_ch), jnp.float32)], axis=0)
    src = edge_index[0].astype(jnp.int32)
    dst = edge_index[1].astype(jnp.int32)
    pad = e_pad - e
    src_p = jnp.concatenate([src, jnp.zeros((pad,), jnp.int32)])
    dst_p = jnp.concatenate([dst, jnp.full((pad,), n, jnp.int32)])

    att2 = att.reshape(1, out_ch)
    blin2 = b_lin.reshape(1, out_ch)
    bias2 = bias_gnn.reshape(1, out_ch)

    grid = (n_pad // ROW_BLK,)
    row_spec = pl.BlockSpec((ROW_BLK, out_ch), lambda i: (i, 0))
    one_spec = pl.BlockSpec((ROW_BLK, 1), lambda i: (i, 0))
    wide_spec = pl.BlockSpec((ROW_BLK, TAB_W), lambda i: (i, 0))
    full_spec = lambda r, c: pl.BlockSpec((r, c), lambda i: (0, 0))

    tab, lin, snum, sden = pl.pallas_call(
        _pre_body,
        grid=grid,
        in_specs=[
            pl.BlockSpec((ROW_BLK, in_ch), lambda i: (i, 0)),
            full_spec(in_ch, out_ch),
            full_spec(in_ch, out_ch),
            full_spec(in_ch, out_ch),
            full_spec(1, out_ch),
            full_spec(1, out_ch),
        ],
        out_specs=[wide_spec, row_spec, row_spec, one_spec],
        out_shape=[
            jax.ShapeDtypeStruct((n_pad, TAB_W), jnp.float32),
            jax.ShapeDtypeStruct((n_pad, out_ch), jnp.float32),
            jax.ShapeDtypeStruct((n_pad, out_ch), jnp.float32),
            jax.ShapeDtypeStruct((n_pad, 1), jnp.float32),
        ],
    )(node_pad, W_l.T, W_r.T, W_lin.T, att2, blin2)

    n_chunks_tot = e_pad // E_CHUNK
    ei_pad = jnp.stack([src_p.reshape(n_chunks_tot, E_CHUNK),
                        dst_p.reshape(n_chunks_tot, E_CHUNK)], axis=1)
    edge_fn = _make_sc_kernel(n_pad, n_acc, out_ch, chunks_per_worker)
    acc_all = edge_fn(tab, ei_pad, att)

    out = pl.pallas_call(
        _post_body,
        grid=grid,
        in_specs=[wide_spec, wide_spec, row_spec, one_spec, row_spec,
                  full_spec(1, out_ch)],
        out_specs=row_spec,
        out_shape=jax.ShapeDtypeStruct((n_pad, out_ch), jnp.float32),
    )(acc_all[0], acc_all[1], snum, sden, lin, bias2)

    return out[:n]


# speedup vs baseline: 1.3253x; 1.0017x over previous
import dataclasses
import functools

import jax
import jax.numpy as jnp
from jax import lax
from jax.experimental import pallas as pl
from jax.experimental.pallas import tpu as pltpu
from jax.experimental.pallas import tpu_sc as plsc

LANES = 16
E_CHUNK = 80
N_WORKERS = 32
NEG_SLOPE = 0.2
ROW_BLK = 1024
TAB_W = 128


def _pre_body(node_ref, wl_ref, wr_ref, wlin_ref, att_ref, blin_ref,
              tab_ref, lin_ref, snum_ref, sden_ref):
    n = node_ref[...]
    xl = jnp.dot(n, wl_ref[...], preferred_element_type=jnp.float32)
    xr = jnp.dot(n, wr_ref[...], preferred_element_type=jnp.float32)
    lin = jnp.dot(n, wlin_ref[...], preferred_element_type=jnp.float32)
    tab_ref[...] = jnp.concatenate([xl, xr], axis=1)
    lin_ref[...] = lin + blin_ref[...]
    t = xl + xr
    leaky = jnp.maximum(t, NEG_SLOPE * t)
    w = jnp.exp(jnp.sum(leaky * att_ref[...], axis=1, keepdims=True))
    snum_ref[...] = xl * w
    sden_ref[...] = w


def _post_body(acc0_ref, acc1_ref, snum_ref, sden_ref, lin_ref, bias_ref,
               out_ref):
    out_ch = out_ref.shape[1]
    a0 = acc0_ref[...]
    a1 = acc1_ref[...]
    num = a0[:, :out_ch] + a1[:, :out_ch] + snum_ref[...]
    den = (a0[:, out_ch:out_ch + 1] + a1[:, out_ch:out_ch + 1]
           + sden_ref[...] + 1e-16)
    out_ref[...] = jnp.maximum(num / den + bias_ref[...] + lin_ref[...], 0.0)


def _make_sc_kernel(n_pad, n_acc, out_ch, chunks_per_worker):
    rows_per_sub = n_acc // 16
    cregs = out_ch // LANES

    def body(tab_hbm, ei_hbm, att_hbm, acc_out,
             i0, i1, sd0, sd1,
             a0, b0, a1, b1, att_v, lg_v, w_v, acc_sh,
             isem0, isem1, ssem0, ssem1):
        cid = lax.axis_index("c")
        sid = lax.axis_index("s")
        zero16 = jnp.zeros((LANES,), jnp.float32)
        lane0 = jnp.arange(LANES) == 0
        n_chunks = chunks_per_worker * N_WORKERS

        @pl.loop(0, E_CHUNK)
        def _(i):
            for c in range(TAB_W // LANES):
                a0[i, pl.ds(c * LANES, LANES)] = zero16

        row0 = sid * rows_per_sub
        full_chunks = (rows_per_sub // E_CHUNK) * E_CHUNK
        tail = rows_per_sub - full_chunks

        @pl.loop(0, full_chunks, step=E_CHUNK)
        def _(r):
            pltpu.sync_copy(a0, acc_sh.at[pl.ds(row0 + r, E_CHUNK)])
        if tail:
            pltpu.sync_copy(a0.at[pl.ds(0, tail)],
                            acc_sh.at[pl.ds(row0 + full_chunks, tail)])

        pltpu.sync_copy(att_hbm, att_v)
        plsc.subcore_barrier()

        att_regs = [att_v[pl.ds(c * LANES, LANES)] for c in range(cregs)]
        wid = cid * 16 + sid
        cbase = wid * chunks_per_worker

        def issue_idx(t, ibuf, isem):
            ci = jnp.minimum(cbase + t, n_chunks - 1)
            pltpu.async_copy(ei_hbm.at[ci], ibuf, isem)

        def wait_idx(ibuf, isem):
            pltpu.make_async_copy(ei_hbm.at[0], ibuf, isem).wait()

        half = E_CHUNK // 2

        def issue_gathers(ibuf, a, b, isem):
            pltpu.async_copy(tab_hbm.at[ibuf.at[0].at[pl.ds(0, half)]],
                             a.at[pl.ds(0, half)], isem)
            pltpu.async_copy(tab_hbm.at[ibuf.at[0].at[pl.ds(half, half)]],
                             a.at[pl.ds(half, half)], isem)
            pltpu.async_copy(tab_hbm.at[ibuf.at[1].at[pl.ds(0, half)]],
                             b.at[pl.ds(0, half)], isem)
            pltpu.async_copy(tab_hbm.at[ibuf.at[1].at[pl.ds(half, half)]],
                             b.at[pl.ds(half, half)], isem)

        def wait_gathers(ibuf, a, b, isem):
            pltpu.make_async_copy(tab_hbm.at[ibuf.at[0].at[pl.ds(0, half)]],
                                  a.at[pl.ds(0, half)], isem).wait()
            pltpu.make_async_copy(tab_hbm.at[ibuf.at[0].at[pl.ds(half, half)]],
                                  a.at[pl.ds(half, half)], isem).wait()
            pltpu.make_async_copy(tab_hbm.at[ibuf.at[1].at[pl.ds(0, half)]],
                                  b.at[pl.ds(0, half)], isem).wait()
            pltpu.make_async_copy(tab_hbm.at[ibuf.at[1].at[pl.ds(half, half)]],
                                  b.at[pl.ds(half, half)], isem).wait()

        def wait_scatter(sd, msg, ssem):
            pltpu.make_async_copy(msg, acc_sh.at[sd], ssem).wait()

        def compute_chunk(a_v, b_v):
            @plsc.parallel_loop(0, E_CHUNK, 1, unroll=8)
            def _(e):
                s = None
                for c in range(cregs):
                    sl = pl.ds(c * LANES, LANES)
                    slr = pl.ds(out_ch + c * LANES, LANES)
                    t_ = a_v[e, sl] + b_v[e, slr]
                    l_ = jnp.maximum(t_, NEG_SLOPE * t_) * att_regs[c]
                    s = l_ if s is None else s + l_
                logit = jnp.sum(s)
                eidx = jnp.full((LANES,), e, jnp.int32)
                plsc.store_scatter(lg_v, [eidx], zero16 + logit, mask=lane0)

            @pl.loop(0, E_CHUNK, step=LANES)
            def _(j):
                w_v[pl.ds(j, LANES)] = jnp.exp(lg_v[pl.ds(j, LANES)])

            @plsc.parallel_loop(0, E_CHUNK, 1, unroll=8)
            def _(e):
                eidx = jnp.full((LANES,), e, jnp.int32)
                wb = plsc.load_gather(w_v, [eidx])
                for c in range(cregs):
                    sl = pl.ds(c * LANES, LANES)
                    a_v[e, sl] = a_v[e, sl] * wb
                a_v[e, pl.ds(out_ch, LANES)] = wb

        def stash_didx(ibuf, sd):
            @pl.loop(0, E_CHUNK, step=LANES)
            def _(j):
                sd[pl.ds(j, LANES)] = ibuf[1, pl.ds(j, LANES)]

        issue_idx(0, i0, isem0)
        issue_idx(1, i1, isem1)
        wait_idx(i0, isem0)
        issue_gathers(i0, a0, b0, isem0)

        @pl.loop(0, chunks_per_worker, step=2)
        def _(t):
            wait_idx(i1, isem1)
            wait_gathers(i0, a0, b0, isem0)

            @pl.when(t > 0)
            def _():
                wait_scatter(sd1, a1, ssem1)
            issue_gathers(i1, a1, b1, isem1)

            stash_didx(i0, sd0)
            issue_idx(t + 2, i0, isem0)
            compute_chunk(a0, b0)
            pltpu.async_copy(a0, acc_sh.at[sd0], ssem0, add=True)

            wait_gathers(i1, a1, b1, isem1)
            stash_didx(i1, sd1)
            issue_idx(t + 3, i1, isem1)
            compute_chunk(a1, b1)
            pltpu.async_copy(a1, acc_sh.at[sd1], ssem1, add=True)

            wait_idx(i0, isem0)
            wait_scatter(sd0, a0, ssem0)
            issue_gathers(i0, a0, b0, isem0)

        wait_gathers(i0, a0, b0, isem0)
        wait_idx(i1, isem1)
        wait_scatter(sd1, a1, ssem1)

        plsc.subcore_barrier()
        pltpu.sync_copy(acc_sh.at[pl.ds(row0, rows_per_sub)],
                        acc_out.at[cid].at[pl.ds(row0, rows_per_sub)])

    mesh = plsc.VectorSubcoreMesh(core_axis_name="c", subcore_axis_name="s")
    cp = pltpu.CompilerParams()
    if "needs_layout_passes" in pltpu.CompilerParams.__dataclass_fields__:
        cp = dataclasses.replace(cp, needs_layout_passes=False)
    return pl.kernel(
        body,
        compiler_params=cp,
        out_type=jax.ShapeDtypeStruct((2, n_pad, TAB_W), jnp.float32),
        mesh=mesh,
        scratch_types=[
            pltpu.VMEM((2, E_CHUNK), jnp.int32),
            pltpu.VMEM((2, E_CHUNK), jnp.int32),
            pltpu.VMEM((E_CHUNK,), jnp.int32),
            pltpu.VMEM((E_CHUNK,), jnp.int32),
            pltpu.VMEM((E_CHUNK, TAB_W), jnp.float32),
            pltpu.VMEM((E_CHUNK, TAB_W), jnp.float32),
            pltpu.VMEM((E_CHUNK, TAB_W), jnp.float32),
            pltpu.VMEM((E_CHUNK, TAB_W), jnp.float32),
            pltpu.VMEM((out_ch,), jnp.float32),
            pltpu.VMEM((E_CHUNK,), jnp.float32),
            pltpu.VMEM((E_CHUNK,), jnp.float32),
            pltpu.VMEM_SHARED((n_acc, TAB_W), jnp.float32),
            pltpu.SemaphoreType.DMA,
            pltpu.SemaphoreType.DMA,
            pltpu.SemaphoreType.DMA,
            pltpu.SemaphoreType.DMA,
        ],
    )


def kernel(node, edge_index, W_l, W_r, att, bias_gnn, W_lin, b_lin):
    n, in_ch = node.shape
    out_ch = W_l.shape[0]
    e = edge_index.shape[1]

    grain = 2 * N_WORKERS * E_CHUNK
    e_pad = ((e + grain - 1) // grain) * grain
    chunks_per_worker = e_pad // (N_WORKERS * E_CHUNK)
    n_pad = ((max(n + 1, ROW_BLK) + (16 * E_CHUNK) - 1)
             // (16 * E_CHUNK)) * (16 * E_CHUNK)
    n_acc = ((n + 1 + 16 * 8 - 1) // (16 * 8)) * (16 * 8)

    node_pad = jnp.concatenate(
        [node, jnp.zeros((n_pad - n, in_ch), jnp.float32)], axis=0)
    src = edge_index[0].astype(jnp.int32)
    dst = edge_index[1].astype(jnp.int32)
    pad = e_pad - e
    src_p = jnp.concatenate([src, jnp.zeros((pad,), jnp.int32)])
    dst_p = jnp.concatenate([dst, jnp.full((pad,), n, jnp.int32)])

    att2 = att.reshape(1, out_ch)
    blin2 = b_lin.reshape(1, out_ch)
    bias2 = bias_gnn.reshape(1, out_ch)

    grid = (n_pad // ROW_BLK,)
    row_spec = pl.BlockSpec((ROW_BLK, out_ch), lambda i: (i, 0))
    one_spec = pl.BlockSpec((ROW_BLK, 1), lambda i: (i, 0))
    wide_spec = pl.BlockSpec((ROW_BLK, TAB_W), lambda i: (i, 0))
    full_spec = lambda r, c: pl.BlockSpec((r, c), lambda i: (0, 0))

    tab, lin, snum, sden = pl.pallas_call(
        _pre_body,
        grid=grid,
        in_specs=[
            pl.BlockSpec((ROW_BLK, in_ch), lambda i: (i, 0)),
            full_spec(in_ch, out_ch),
            full_spec(in_ch, out_ch),
            full_spec(in_ch, out_ch),
            full_spec(1, out_ch),
            full_spec(1, out_ch),
        ],
        out_specs=[wide_spec, row_spec, row_spec, one_spec],
        out_shape=[
            jax.ShapeDtypeStruct((n_pad, TAB_W), jnp.float32),
            jax.ShapeDtypeStruct((n_pad, out_ch), jnp.float32),
            jax.ShapeDtypeStruct((n_pad, out_ch), jnp.float32),
            jax.ShapeDtypeStruct((n_pad, 1), jnp.float32),
        ],
    )(node_pad, W_l.T, W_r.T, W_lin.T, att2, blin2)

    n_chunks_tot = e_pad // E_CHUNK
    ei_pad = jnp.stack([src_p.reshape(n_chunks_tot, E_CHUNK),
                        dst_p.reshape(n_chunks_tot, E_CHUNK)], axis=1)
    edge_fn = _make_sc_kernel(n_pad, n_acc, out_ch, chunks_per_worker)
    acc_all = edge_fn(tab, ei_pad, att)

    out = pl.pallas_call(
        _post_body,
        grid=grid,
        in_specs=[wide_spec, wide_spec, row_spec, one_spec, row_spec,
                  full_spec(1, out_ch)],
        out_specs=row_spec,
        out_shape=jax.ShapeDtypeStruct((n_pad, out_ch), jnp.float32),
    )(acc_all[0], acc_all[1], snum, sden, lin, bias2)

    return out[:n]


# speedup vs baseline: 1.7018x; 1.2841x over previous
import dataclasses
import functools

import jax
import jax.numpy as jnp
from jax import lax
from jax.experimental import pallas as pl
from jax.experimental.pallas import tpu as pltpu
from jax.experimental.pallas import tpu_sc as plsc

LANES = 16
E_CHUNK = 80
N_WORKERS = 32
NEG_SLOPE = 0.2
ROW_BLK = 1024
TAB_W = 128


def _pre_body(node_ref, wl_ref, wr_ref, wlin_ref, att_ref, blin_ref,
              tab_ref, tabr_ref, lin_ref, snum_ref, sden_ref):
    n = node_ref[...]
    xl = jnp.dot(n, wl_ref[...], preferred_element_type=jnp.float32)
    xr = jnp.dot(n, wr_ref[...], preferred_element_type=jnp.float32)
    lin = jnp.dot(n, wlin_ref[...], preferred_element_type=jnp.float32)
    tab_ref[...] = xl
    tabr_ref[...] = xr
    lin_ref[...] = lin + blin_ref[...]
    t = xl + xr
    leaky = jnp.maximum(t, NEG_SLOPE * t)
    w = jnp.exp(jnp.sum(leaky * att_ref[...], axis=1, keepdims=True))
    snum_ref[...] = xl * w
    sden_ref[...] = w


def _post_body(acc0_ref, acc1_ref, snum_ref, sden_ref, lin_ref, bias_ref,
               out_ref):
    out_ch = out_ref.shape[1]
    a0 = acc0_ref[...]
    a1 = acc1_ref[...]
    num = a0[:, :out_ch] + a1[:, :out_ch] + snum_ref[...]
    den = (a0[:, out_ch:out_ch + 1] + a1[:, out_ch:out_ch + 1]
           + sden_ref[...] + 1e-16)
    out_ref[...] = jnp.maximum(num / den + bias_ref[...] + lin_ref[...], 0.0)


def _make_sc_kernel(n_pad, n_acc, out_ch, chunks_per_worker):
    rows_per_sub = n_acc // 16
    cregs = out_ch // LANES

    def body(tab_hbm, tabr_hbm, ei_hbm, att_hbm, acc_out,
             i0, i1, sd0, sd1,
             a0, b0, a1, b1, msg0, msg1, att_v, lg_v, w_v, acc_sh,
             isem0, isem1, ssem0, ssem1):
        cid = lax.axis_index("c")
        sid = lax.axis_index("s")
        zero16 = jnp.zeros((LANES,), jnp.float32)
        lane0 = jnp.arange(LANES) == 0
        n_chunks = chunks_per_worker * N_WORKERS

        @pl.loop(0, E_CHUNK)
        def _(i):
            for c in range(TAB_W // LANES):
                msg0[i, pl.ds(c * LANES, LANES)] = zero16
            for c in range(TAB_W // LANES):
                msg1[i, pl.ds(c * LANES, LANES)] = zero16

        row0 = sid * rows_per_sub
        full_chunks = (rows_per_sub // E_CHUNK) * E_CHUNK
        tail = rows_per_sub - full_chunks

        @pl.loop(0, full_chunks, step=E_CHUNK)
        def _(r):
            pltpu.sync_copy(msg0, acc_sh.at[pl.ds(row0 + r, E_CHUNK)])
        if tail:
            pltpu.sync_copy(msg0.at[pl.ds(0, tail)],
                            acc_sh.at[pl.ds(row0 + full_chunks, tail)])

        pltpu.sync_copy(att_hbm, att_v)
        plsc.subcore_barrier()

        att_regs = [att_v[pl.ds(c * LANES, LANES)] for c in range(cregs)]
        wid = cid * 16 + sid
        cbase = wid * chunks_per_worker

        def issue_idx(t, ibuf, isem):
            ci = jnp.minimum(cbase + t, n_chunks - 1)
            pltpu.async_copy(ei_hbm.at[ci], ibuf, isem)

        def wait_idx(ibuf, isem):
            pltpu.make_async_copy(ei_hbm.at[0], ibuf, isem).wait()

        def issue_gathers(ibuf, a, b, isem):
            pltpu.async_copy(tab_hbm.at[ibuf.at[0]], a, isem)
            pltpu.async_copy(tabr_hbm.at[ibuf.at[1]], b, isem)

        def wait_gathers(ibuf, a, b, isem):
            pltpu.make_async_copy(tab_hbm.at[ibuf.at[0]], a, isem).wait()
            pltpu.make_async_copy(tabr_hbm.at[ibuf.at[1]], b, isem).wait()

        def wait_scatter(sd, msg, ssem):
            pltpu.make_async_copy(msg, acc_sh.at[sd], ssem).wait()

        def pass_a(a_v, b_v):
            @plsc.parallel_loop(0, E_CHUNK, 1, unroll=8)
            def _(e):
                s = None
                for c in range(cregs):
                    sl = pl.ds(c * LANES, LANES)
                    t_ = a_v[e, sl] + b_v[e, sl]
                    l_ = jnp.maximum(t_, NEG_SLOPE * t_) * att_regs[c]
                    s = l_ if s is None else s + l_
                logit = jnp.sum(s)
                eidx = jnp.full((LANES,), e, jnp.int32)
                plsc.store_scatter(lg_v, [eidx], zero16 + logit, mask=lane0)

            @pl.loop(0, E_CHUNK, step=LANES)
            def _(j):
                w_v[pl.ds(j, LANES)] = jnp.exp(lg_v[pl.ds(j, LANES)])

        def pass_b(a_v, msg_v):
            @plsc.parallel_loop(0, E_CHUNK, 1, unroll=8)
            def _(e):
                eidx = jnp.full((LANES,), e, jnp.int32)
                wb = plsc.load_gather(w_v, [eidx])
                for c in range(cregs):
                    sl = pl.ds(c * LANES, LANES)
                    msg_v[e, sl] = a_v[e, sl] * wb
                msg_v[e, pl.ds(out_ch, LANES)] = wb

        def stash_didx(ibuf, sd):
            @pl.loop(0, E_CHUNK, step=LANES)
            def _(j):
                sd[pl.ds(j, LANES)] = ibuf[1, pl.ds(j, LANES)]

        issue_idx(0, i0, isem0)
        issue_idx(1, i1, isem1)
        wait_idx(i0, isem0)
        issue_gathers(i0, a0, b0, isem0)

        @pl.loop(0, chunks_per_worker, step=2)
        def _(t):
            wait_idx(i1, isem1)
            wait_gathers(i0, a0, b0, isem0)
            issue_gathers(i1, a1, b1, isem1)

            stash_didx(i0, sd0)
            issue_idx(t + 2, i0, isem0)
            pass_a(a0, b0)
            @pl.when(t > 0)
            def _():
                wait_scatter(sd0, msg0, ssem0)
            pass_b(a0, msg0)
            pltpu.async_copy(msg0, acc_sh.at[sd0], ssem0, add=True)

            wait_gathers(i1, a1, b1, isem1)
            stash_didx(i1, sd1)
            issue_idx(t + 3, i1, isem1)
            pass_a(a1, b1)
            @pl.when(t > 0)
            def _():
                wait_scatter(sd1, msg1, ssem1)
            pass_b(a1, msg1)
            pltpu.async_copy(msg1, acc_sh.at[sd1], ssem1, add=True)

            wait_idx(i0, isem0)
            issue_gathers(i0, a0, b0, isem0)

        wait_gathers(i0, a0, b0, isem0)
        wait_idx(i1, isem1)
        wait_scatter(sd0, msg0, ssem0)
        wait_scatter(sd1, msg1, ssem1)

        plsc.subcore_barrier()
        pltpu.sync_copy(acc_sh.at[pl.ds(row0, rows_per_sub)],
                        acc_out.at[cid].at[pl.ds(row0, rows_per_sub)])

    mesh = plsc.VectorSubcoreMesh(core_axis_name="c", subcore_axis_name="s")
    cp = pltpu.CompilerParams(use_tc_tiling_on_sc=False)
    if "needs_layout_passes" in pltpu.CompilerParams.__dataclass_fields__:
        cp = dataclasses.replace(cp, needs_layout_passes=False)
    return pl.kernel(
        body,
        compiler_params=cp,
        out_type=jax.ShapeDtypeStruct((2, n_pad, TAB_W), jnp.float32),
        mesh=mesh,
        scratch_types=[
            pltpu.VMEM((2, E_CHUNK), jnp.int32),
            pltpu.VMEM((2, E_CHUNK), jnp.int32),
            pltpu.VMEM((E_CHUNK,), jnp.int32),
            pltpu.VMEM((E_CHUNK,), jnp.int32),
            pltpu.VMEM((E_CHUNK, out_ch), jnp.float32),
            pltpu.VMEM((E_CHUNK, out_ch), jnp.float32),
            pltpu.VMEM((E_CHUNK, out_ch), jnp.float32),
            pltpu.VMEM((E_CHUNK, out_ch), jnp.float32),
            pltpu.VMEM((E_CHUNK, TAB_W), jnp.float32),
            pltpu.VMEM((E_CHUNK, TAB_W), jnp.float32),
            pltpu.VMEM((out_ch,), jnp.float32),
            pltpu.VMEM((E_CHUNK,), jnp.float32),
            pltpu.VMEM((E_CHUNK,), jnp.float32),
            pltpu.VMEM_SHARED((n_acc, TAB_W), jnp.float32),
            pltpu.SemaphoreType.DMA,
            pltpu.SemaphoreType.DMA,
            pltpu.SemaphoreType.DMA,
            pltpu.SemaphoreType.DMA,
        ],
    )


def kernel(node, edge_index, W_l, W_r, att, bias_gnn, W_lin, b_lin):
    n, in_ch = node.shape
    out_ch = W_l.shape[0]
    e = edge_index.shape[1]

    grain = 2 * N_WORKERS * E_CHUNK
    e_pad = ((e + grain - 1) // grain) * grain
    chunks_per_worker = e_pad // (N_WORKERS * E_CHUNK)
    n_pad = ((max(n + 1, ROW_BLK) + (16 * E_CHUNK) - 1)
             // (16 * E_CHUNK)) * (16 * E_CHUNK)
    n_acc = ((n + 1 + 16 * 8 - 1) // (16 * 8)) * (16 * 8)

    node_pad = jnp.concatenate(
        [node, jnp.zeros((n_pad - n, in_ch), jnp.float32)], axis=0)
    src = edge_index[0].astype(jnp.int32)
    dst = edge_index[1].astype(jnp.int32)
    pad = e_pad - e
    src_p = jnp.concatenate([src, jnp.zeros((pad,), jnp.int32)])
    dst_p = jnp.concatenate([dst, jnp.full((pad,), n, jnp.int32)])

    att2 = att.reshape(1, out_ch)
    blin2 = b_lin.reshape(1, out_ch)
    bias2 = bias_gnn.reshape(1, out_ch)

    grid = (n_pad // ROW_BLK,)
    row_spec = pl.BlockSpec((ROW_BLK, out_ch), lambda i: (i, 0))
    one_spec = pl.BlockSpec((ROW_BLK, 1), lambda i: (i, 0))
    wide_spec = pl.BlockSpec((ROW_BLK, TAB_W), lambda i: (i, 0))
    full_spec = lambda r, c: pl.BlockSpec((r, c), lambda i: (0, 0))

    tab, tabr, lin, snum, sden = pl.pallas_call(
        _pre_body,
        grid=grid,
        in_specs=[
            pl.BlockSpec((ROW_BLK, in_ch), lambda i: (i, 0)),
            full_spec(in_ch, out_ch),
            full_spec(in_ch, out_ch),
            full_spec(in_ch, out_ch),
            full_spec(1, out_ch),
            full_spec(1, out_ch),
        ],
        out_specs=[row_spec, row_spec, row_spec, row_spec, one_spec],
        out_shape=[
            jax.ShapeDtypeStruct((n_pad, out_ch), jnp.float32),
            jax.ShapeDtypeStruct((n_pad, out_ch), jnp.float32),
            jax.ShapeDtypeStruct((n_pad, out_ch), jnp.float32),
            jax.ShapeDtypeStruct((n_pad, out_ch), jnp.float32),
            jax.ShapeDtypeStruct((n_pad, 1), jnp.float32),
        ],
    )(node_pad, W_l.T, W_r.T, W_lin.T, att2, blin2)

    n_chunks_tot = e_pad // E_CHUNK
    ei_pad = jnp.stack([src_p.reshape(n_chunks_tot, E_CHUNK),
                        dst_p.reshape(n_chunks_tot, E_CHUNK)], axis=1)
    edge_fn = _make_sc_kernel(n_pad, n_acc, out_ch, chunks_per_worker)
    acc_all = edge_fn(tab, tabr, ei_pad, att)

    out = pl.pallas_call(
        _post_body,
        grid=grid,
        in_specs=[wide_spec, wide_spec, row_spec, one_spec, row_spec,
                  full_spec(1, out_ch)],
        out_specs=row_spec,
        out_shape=jax.ShapeDtypeStruct((n_pad, out_ch), jnp.float32),
    )(acc_all[0], acc_all[1], snum, sden, lin, bias2)

    return out[:n]
